# baseline (speedup 1.0000x reference)
import functools

import jax
import jax.numpy as jnp
from jax import lax
from jax.experimental import pallas as pl
from jax.experimental.pallas import tpu as pltpu
from jax.experimental.pallas import tpu_sc as plsc

N = 2048
E = 32768
H = 1024
IN_CH = 128
HID = 64
HID2 = 2 * HID
LAT = 32

NC = 2
NS = 16
NW = NC * NS
EPW = E // NW
CH = 128
NCH = EPW // CH
RPT = N // NS

_sc_mesh = plsc.VectorSubcoreMesh(core_axis_name="c", subcore_axis_name="s")
_sc_params = pltpu.CompilerParams(use_tc_tiling_on_sc=True)


@functools.partial(
    pl.kernel,
    out_type=jax.ShapeDtypeStruct((NC, N), jnp.float32),
    mesh=_sc_mesh,
    scratch_types=[
        pltpu.VMEM((NCH, CH), jnp.int32),
        pltpu.VMEM((CH,), jnp.float32),
        pltpu.VMEM((RPT,), jnp.float32),
        pltpu.VMEM_SHARED((N,), jnp.float32),
        pltpu.SemaphoreType.DMA,
    ],
    compiler_params=_sc_params,
)
def _hist_kernel(col_hbm, out_hbm, idx_v, ones_v, half_v, hist_sh, sem):
    cid = lax.axis_index("c")
    sid = lax.axis_index("s")
    wid = sid * NC + cid
    base = wid * EPW
    descs = []
    for j in range(NCH):
        descs.append(
            pltpu.async_copy(col_hbm.at[pl.ds(base + j * CH, CH)],
                             idx_v.at[j], sem))
    for k in range(CH // 16):
        ones_v[pl.ds(k * 16, 16)] = jnp.full((16,), 1.0, jnp.float32)
    for k in range(RPT // 16):
        half_v[pl.ds(k * 16, 16)] = jnp.full((16,), 0.5, jnp.float32)
    pltpu.sync_copy(half_v, hist_sh.at[pl.ds(sid * RPT, RPT)])
    for d in descs:
        d.wait()
    plsc.subcore_barrier()
    for j in range(NCH):
        pltpu.sync_copy(ones_v, hist_sh.at[idx_v.at[j]], add=True)
    plsc.subcore_barrier()
    pltpu.sync_copy(
        hist_sh.at[pl.ds(sid * RPT, RPT)], out_hbm.at[cid, pl.ds(sid * RPT, RPT)]
    )


_RB = 256
_NSTEPS = H // _RB


def _ka_body(my_ref, a_ref, cs_ref, csr_ref):
    i = pl.program_id(0)
    v = my_ref[...]
    a = jnp.where(v != 0.0, jax.nn.sigmoid(v), 0.0)
    a_ref[...] = a
    ones_rb = jnp.ones((_RB, 1), jnp.float32)
    part = lax.dot_general(a, ones_rb, (((0,), (0,)), ((), ())),
                           preferred_element_type=jnp.float32)
    part_row = jnp.sum(a, axis=0, keepdims=True)

    @pl.when(i == 0)
    def _():
        cs_ref[...] = part
        csr_ref[...] = part_row

    @pl.when(i != 0)
    def _():
        cs_ref[...] = cs_ref[...] + part
        csr_ref[...] = csr_ref[...] + part_row


def _make_a(my):
    return pl.pallas_call(
        _ka_body,
        grid=(_NSTEPS,),
        in_specs=[pl.BlockSpec((_RB, H), lambda i: (i, 0))],
        out_specs=[
            pl.BlockSpec((_RB, H), lambda i: (i, 0)),
            pl.BlockSpec((H, 1), lambda i: (0, 0)),
            pl.BlockSpec((1, H), lambda i: (0, 0)),
        ],
        out_shape=[
            jax.ShapeDtypeStruct((H, H), jnp.float32),
            jax.ShapeDtypeStruct((H, 1), jnp.float32),
            jax.ShapeDtypeStruct((1, H), jnp.float32),
        ],
    )(my)


def _ks1_body(cs_ref, csr_ref, h_ref, x_ref, w1_ref, dis_ref, disr_ref, s1_ref):
    ones2 = jnp.ones((2, 1), jnp.float32)
    h_col = lax.dot_general(h_ref[...], ones2, (((0,), (0,)), ((), ())),
                            preferred_element_type=jnp.float32)
    deg_top = cs_ref[...] + h_col[:H, :]
    deg_bot = h_col[H:, :]
    dis = lax.rsqrt(jnp.concatenate([deg_top, deg_bot], axis=0))
    dis_ref[...] = dis
    h_row = jnp.sum(h_ref[...], axis=0, keepdims=True)
    deg_row = jnp.concatenate(
        [csr_ref[...], jnp.zeros((1, N - H), jnp.float32)], axis=1) + h_row
    disr_ref[...] = lax.rsqrt(deg_row)
    xw = jnp.dot(x_ref[...], w1_ref[...], preferred_element_type=jnp.float32)
    s = dis * xw
    s1_ref[...] = jnp.concatenate([s, 0.5 * s], axis=1)


def _make_s1(cs, csr, histp, x, w1):
    return pl.pallas_call(
        _ks1_body,
        out_shape=[
            jax.ShapeDtypeStruct((N, 1), jnp.float32),
            jax.ShapeDtypeStruct((1, N), jnp.float32),
            jax.ShapeDtypeStruct((N, HID2), jnp.float32),
        ],
    )(cs, csr, histp, x, w1)


def _kt_body(a_ref, sp_ref, t_ref):
    t_ref[...] = lax.dot_general(a_ref[...], sp_ref[:H, :HID],
                                 (((0,), (0,)), ((), ())),
                                 preferred_element_type=jnp.float32)


def _make_t(a, sp):
    return pl.pallas_call(
        _kt_body,
        out_shape=jax.ShapeDtypeStruct((H, HID), jnp.float32),
    )(a, sp)


def _ktt_body(a_ref, sp_ref, t_ref):
    t_ref[...] = lax.dot_general(sp_ref[:H, :HID], a_ref[...],
                                 (((0,), (0,)), ((), ())),
                                 preferred_element_type=jnp.float32)


def _make_tT(a, sp):
    return pl.pallas_call(
        _ktt_body,
        out_shape=jax.ShapeDtypeStruct((HID, H), jnp.float32),
    )(a, sp)


def _scat_body(sp_hbm, row_hbm, col_hbm, out_hbm,
               ridx_v, cidx_v, rows_v, u_sh, sem_i, sem_ld, sem_g):
    cid = lax.axis_index("c")
    sid = lax.axis_index("s")
    wid = sid * NC + cid
    base = wid * EPW
    descs = []
    for j in range(NCH):
        descs.append(
            pltpu.async_copy(row_hbm.at[pl.ds(base + j * CH, CH)],
                             ridx_v.at[j], sem_ld))
        descs.append(
            pltpu.async_copy(col_hbm.at[pl.ds(base + j * CH, CH)],
                             cidx_v.at[j], sem_ld))
    d_init = pltpu.async_copy(sp_hbm.at[pl.ds(sid * RPT, RPT)],
                              u_sh.at[pl.ds(sid * RPT, RPT)], sem_i)
    for d in descs:
        d.wait()
    gs = [pltpu.async_copy(sp_hbm.at[ridx_v.at[j]], rows_v.at[j % _BUF], sem_g)
          for j in range(_BUF - 1)]
    d_init.wait()
    plsc.subcore_barrier()
    for j in range(NCH):
        gs[j].wait()
        if j + _BUF - 1 < NCH:
            gs.append(
                pltpu.async_copy(sp_hbm.at[ridx_v.at[j + _BUF - 1]],
                                 rows_v.at[(j + _BUF - 1) % _BUF], sem_g))
        pltpu.sync_copy(rows_v.at[j % _BUF], u_sh.at[cidx_v.at[j]], add=True)
    plsc.subcore_barrier()
    pltpu.sync_copy(u_sh.at[pl.ds(sid * RPT, RPT)],
                    out_hbm.at[cid, pl.ds(sid * RPT, RPT)])


_BUF = 3

_scatter = pl.kernel(
    _scat_body,
    out_type=jax.ShapeDtypeStruct((NC, N, HID2), jnp.float32),
    mesh=_sc_mesh,
    scratch_types=[
        pltpu.VMEM((NCH, CH), jnp.int32),
        pltpu.VMEM((NCH, CH), jnp.int32),
        pltpu.VMEM((_BUF, CH, HID2), jnp.float32),
        pltpu.VMEM_SHARED((N, HID2), jnp.float32),
        pltpu.SemaphoreType.DMA,
        pltpu.SemaphoreType.DMA,
        pltpu.SemaphoreType.DMA,
    ],
    compiler_params=_sc_params,
)


def _kc1_body(t_ref, s1_ref, u_ref, dis_ref, b1_ref, wmu_ref, wls_ref,
              s2_ref):
    s1 = s1_ref[:, :HID]
    t_top = t_ref[...]
    u = u_ref[0, :, :HID] + u_ref[1, :, :HID] - s1
    b1v = b1_ref[...]
    pre_top = dis_ref[:H, :] * (t_top + u[:H, :]) + b1v
    pre_bot = dis_ref[H:, :] * u[H:, :] + b1v
    hid_top = jnp.maximum(pre_top, 0.0)
    hid_bot = jnp.maximum(pre_bot, 0.0)
    wc = jnp.concatenate([wmu_ref[...], wls_ref[...]], axis=1)
    s2_top = dis_ref[:H, :] * jnp.dot(hid_top, wc,
                                      preferred_element_type=jnp.float32)
    s2_bot = dis_ref[H:, :] * jnp.dot(hid_bot, wc,
                                      preferred_element_type=jnp.float32)
    s2_ref[:H, :] = jnp.concatenate([s2_top, 0.5 * s2_top], axis=1)
    s2_ref[H:, :] = jnp.concatenate([s2_bot, 0.5 * s2_bot], axis=1)


def _make_s2(t1, s1, u1, dis, b1, wmu, wls):
    return pl.pallas_call(
        _kc1_body,
        out_shape=jax.ShapeDtypeStruct((N, HID2), jnp.float32),
    )(t1, s1, u1, dis, b1, wmu, wls)


def _ko_body(tt_ref, s2_ref, u_ref, disr_ref, bmu_ref, bls_ref,
             mu_ref, ls_ref):
    s2 = s2_ref[:, :HID]
    u = u_ref[0, :, :HID] + u_ref[1, :, :HID] - s2
    ut = lax.transpose(u, (1, 0))
    left = tt_ref[...] + ut[:, :H]
    ot = disr_ref[...] * jnp.concatenate([left, ut[:, H:]], axis=1)
    mu_ref[...] = ot[:LAT, :] + bmu_ref[...]
    ls_ref[...] = ot[LAT:, :] + bls_ref[...]


def _make_out(t2t, s2, u2, disr, bmu, bls):
    return pl.pallas_call(
        _ko_body,
        out_shape=[
            jax.ShapeDtypeStruct((LAT, N), jnp.float32),
            jax.ShapeDtypeStruct((LAT, N), jnp.float32),
        ],
    )(t2t, s2, u2, disr, bmu, bls)


def kernel(x, edge_index, masked_y, W1, b1, W_mu, b_mu, W_logstd, b_logstd):
    ei = edge_index.astype(jnp.int32)
    row = ei[0]
    col = ei[1]
    histp = _hist_kernel(col)
    a, cs, csr = _make_a(masked_y)
    dis, disr, s1p = _make_s1(cs, csr, histp, x, W1)
    u1 = _scatter(s1p, row, col)
    t1 = _make_t(a, s1p)
    s2p = _make_s2(t1, s1p, u1, dis, b1, W_mu, W_logstd)
    u2 = _scatter(s2p, row, col)
    t2t = _make_tT(a, s2p)
    z_mu_t, z_logstd_t = _make_out(t2t, s2p, u2, disr,
                                   b_mu.reshape(LAT, 1), b_logstd.reshape(LAT, 1))
    return (z_mu_t.T, z_logstd_t.T)


# speedup vs baseline: 1.0330x; 1.0330x over previous
import functools

import jax
import jax.numpy as jnp
from jax import lax
from jax.experimental import pallas as pl
from jax.experimental.pallas import tpu as pltpu
from jax.experimental.pallas import tpu_sc as plsc

N = 2048
E = 32768
H = 1024
IN_CH = 128
HID = 64
HID2 = 2 * HID
LAT = 32

NC = 2
NS = 16
NW = NC * NS
EPW = E // NW
CH = 128
NCH = EPW // CH
RPT = N // NS

_sc_mesh = plsc.VectorSubcoreMesh(core_axis_name="c", subcore_axis_name="s")
_sc_params = pltpu.CompilerParams(use_tc_tiling_on_sc=True)


@functools.partial(
    pl.kernel,
    out_type=jax.ShapeDtypeStruct((NC, N), jnp.float32),
    mesh=_sc_mesh,
    scratch_types=[
        pltpu.VMEM((NCH, CH), jnp.int32),
        pltpu.VMEM((CH,), jnp.float32),
        pltpu.VMEM((RPT,), jnp.float32),
        pltpu.VMEM_SHARED((N,), jnp.float32),
        pltpu.SemaphoreType.DMA,
    ],
    compiler_params=_sc_params,
)
def _hist_kernel(ei_hbm, out_hbm, idx_v, ones_v, half_v, hist_sh, sem):
    cid = lax.axis_index("c")
    sid = lax.axis_index("s")
    wid = sid * NC + cid
    base = wid * EPW
    descs = []
    for j in range(NCH):
        descs.append(
            pltpu.async_copy(ei_hbm.at[1, pl.ds(base + j * CH, CH)],
                             idx_v.at[j], sem))
    for k in range(CH // 16):
        ones_v[pl.ds(k * 16, 16)] = jnp.full((16,), 1.0, jnp.float32)
    for k in range(RPT // 16):
        half_v[pl.ds(k * 16, 16)] = jnp.full((16,), 0.5, jnp.float32)
    pltpu.sync_copy(half_v, hist_sh.at[pl.ds(sid * RPT, RPT)])
    for d in descs:
        d.wait()
    plsc.subcore_barrier()
    for j in range(NCH):
        pltpu.sync_copy(ones_v, hist_sh.at[idx_v.at[j]], add=True)
    plsc.subcore_barrier()
    pltpu.sync_copy(
        hist_sh.at[pl.ds(sid * RPT, RPT)], out_hbm.at[cid, pl.ds(sid * RPT, RPT)]
    )


_RB = 256
_NSTEPS = H // _RB


def _ka_body(my_ref, a_ref, cs_ref, csr_ref):
    i = pl.program_id(0)
    v = my_ref[...]
    a = jnp.where(v != 0.0, jax.nn.sigmoid(v), 0.0)
    a_ref[...] = a
    ones_rb = jnp.ones((_RB, 1), jnp.float32)
    part = lax.dot_general(a, ones_rb, (((0,), (0,)), ((), ())),
                           preferred_element_type=jnp.float32)
    part_row = jnp.sum(a, axis=0, keepdims=True)

    @pl.when(i == 0)
    def _():
        cs_ref[...] = part
        csr_ref[...] = part_row

    @pl.when(i != 0)
    def _():
        cs_ref[...] = cs_ref[...] + part
        csr_ref[...] = csr_ref[...] + part_row


def _make_a(my):
    return pl.pallas_call(
        _ka_body,
        grid=(_NSTEPS,),
        in_specs=[pl.BlockSpec((_RB, H), lambda i: (i, 0))],
        out_specs=[
            pl.BlockSpec((_RB, H), lambda i: (i, 0)),
            pl.BlockSpec((H, 1), lambda i: (0, 0)),
            pl.BlockSpec((1, H), lambda i: (0, 0)),
        ],
        out_shape=[
            jax.ShapeDtypeStruct((H, H), jnp.float32),
            jax.ShapeDtypeStruct((H, 1), jnp.float32),
            jax.ShapeDtypeStruct((1, H), jnp.float32),
        ],
    )(my)


def _ks1_body(cs_ref, csr_ref, h_ref, x_ref, w1_ref, dis_ref, disr_ref, s1_ref):
    ones2 = jnp.ones((2, 1), jnp.float32)
    h_col = lax.dot_general(h_ref[...], ones2, (((0,), (0,)), ((), ())),
                            preferred_element_type=jnp.float32)
    deg_top = cs_ref[...] + h_col[:H, :]
    deg_bot = h_col[H:, :]
    dis = lax.rsqrt(jnp.concatenate([deg_top, deg_bot], axis=0))
    dis_ref[...] = dis
    h_row = jnp.sum(h_ref[...], axis=0, keepdims=True)
    deg_row = jnp.concatenate(
        [csr_ref[...], jnp.zeros((1, N - H), jnp.float32)], axis=1) + h_row
    disr_ref[...] = lax.rsqrt(deg_row)
    xw = jnp.dot(x_ref[...], w1_ref[...], preferred_element_type=jnp.float32)
    s = dis * xw
    s1_ref[...] = jnp.concatenate([s, 0.5 * s], axis=1)


def _make_s1(cs, csr, histp, x, w1):
    return pl.pallas_call(
        _ks1_body,
        out_shape=[
            jax.ShapeDtypeStruct((N, 1), jnp.float32),
            jax.ShapeDtypeStruct((1, N), jnp.float32),
            jax.ShapeDtypeStruct((N, HID2), jnp.float32),
        ],
    )(cs, csr, histp, x, w1)


def _kt_body(a_ref, sp_ref, t_ref):
    t_ref[...] = lax.dot_general(a_ref[...], sp_ref[:H, :HID],
                                 (((0,), (0,)), ((), ())),
                                 preferred_element_type=jnp.float32)


def _make_t(a, sp):
    return pl.pallas_call(
        _kt_body,
        out_shape=jax.ShapeDtypeStruct((H, HID), jnp.float32),
    )(a, sp)


def _ktt_body(a_ref, sp_ref, t_ref):
    t_ref[...] = lax.dot_general(sp_ref[:H, :HID], a_ref[...],
                                 (((0,), (0,)), ((), ())),
                                 preferred_element_type=jnp.float32)


def _make_tT(a, sp):
    return pl.pallas_call(
        _ktt_body,
        out_shape=jax.ShapeDtypeStruct((HID, H), jnp.float32),
    )(a, sp)


def _scat_body(sp_hbm, ei_hbm, out_hbm,
               ridx_v, cidx_v, rows_v, u_sh, sem_i, sem_ld, sem_g):
    cid = lax.axis_index("c")
    sid = lax.axis_index("s")
    wid = sid * NC + cid
    base = wid * EPW
    descs = []
    for j in range(NCH):
        descs.append(
            pltpu.async_copy(ei_hbm.at[0, pl.ds(base + j * CH, CH)],
                             ridx_v.at[j], sem_ld))
        descs.append(
            pltpu.async_copy(ei_hbm.at[1, pl.ds(base + j * CH, CH)],
                             cidx_v.at[j], sem_ld))
    d_init = pltpu.async_copy(sp_hbm.at[pl.ds(sid * RPT, RPT)],
                              u_sh.at[pl.ds(sid * RPT, RPT)], sem_i)
    for d in descs:
        d.wait()
    gs = [pltpu.async_copy(sp_hbm.at[ridx_v.at[j]], rows_v.at[j % _BUF], sem_g)
          for j in range(_BUF - 1)]
    d_init.wait()
    plsc.subcore_barrier()
    for j in range(NCH):
        gs[j].wait()
        if j + _BUF - 1 < NCH:
            gs.append(
                pltpu.async_copy(sp_hbm.at[ridx_v.at[j + _BUF - 1]],
                                 rows_v.at[(j + _BUF - 1) % _BUF], sem_g))
        pltpu.sync_copy(rows_v.at[j % _BUF], u_sh.at[cidx_v.at[j]], add=True)
    plsc.subcore_barrier()
    pltpu.sync_copy(u_sh.at[pl.ds(sid * RPT, RPT)],
                    out_hbm.at[cid, pl.ds(sid * RPT, RPT)])


_BUF = 3

_scatter = pl.kernel(
    _scat_body,
    out_type=jax.ShapeDtypeStruct((NC, N, HID2), jnp.float32),
    mesh=_sc_mesh,
    scratch_types=[
        pltpu.VMEM((NCH, CH), jnp.int32),
        pltpu.VMEM((NCH, CH), jnp.int32),
        pltpu.VMEM((_BUF, CH, HID2), jnp.float32),
        pltpu.VMEM_SHARED((N, HID2), jnp.float32),
        pltpu.SemaphoreType.DMA,
        pltpu.SemaphoreType.DMA,
        pltpu.SemaphoreType.DMA,
    ],
    compiler_params=_sc_params,
)


def _kc1_body(t_ref, s1_ref, u_ref, dis_ref, b1_ref, wmu_ref, wls_ref,
              s2_ref):
    s1 = s1_ref[:, :HID]
    t_top = t_ref[...]
    u = u_ref[0, :, :HID] + u_ref[1, :, :HID] - s1
    b1v = b1_ref[...]
    pre_top = dis_ref[:H, :] * (t_top + u[:H, :]) + b1v
    pre_bot = dis_ref[H:, :] * u[H:, :] + b1v
    hid_top = jnp.maximum(pre_top, 0.0)
    hid_bot = jnp.maximum(pre_bot, 0.0)
    wc = jnp.concatenate([wmu_ref[...], wls_ref[...]], axis=1)
    s2_top = dis_ref[:H, :] * jnp.dot(hid_top, wc,
                                      preferred_element_type=jnp.float32)
    s2_bot = dis_ref[H:, :] * jnp.dot(hid_bot, wc,
                                      preferred_element_type=jnp.float32)
    s2_ref[:H, :] = jnp.concatenate([s2_top, 0.5 * s2_top], axis=1)
    s2_ref[H:, :] = jnp.concatenate([s2_bot, 0.5 * s2_bot], axis=1)


def _make_s2(t1, s1, u1, dis, b1, wmu, wls):
    return pl.pallas_call(
        _kc1_body,
        out_shape=jax.ShapeDtypeStruct((N, HID2), jnp.float32),
    )(t1, s1, u1, dis, b1, wmu, wls)


def _ko_body(tt_ref, s2_ref, u_ref, disr_ref, bmu_ref, bls_ref,
             mu_ref, ls_ref):
    s2 = s2_ref[:, :HID]
    u = u_ref[0, :, :HID] + u_ref[1, :, :HID] - s2
    ut = lax.transpose(u, (1, 0))
    left = tt_ref[...] + ut[:, :H]
    ot = disr_ref[...] * jnp.concatenate([left, ut[:, H:]], axis=1)
    mu_ref[...] = ot[:LAT, :] + bmu_ref[...]
    ls_ref[...] = ot[LAT:, :] + bls_ref[...]


def _make_out(t2t, s2, u2, disr, bmu, bls):
    return pl.pallas_call(
        _ko_body,
        out_shape=[
            jax.ShapeDtypeStruct((LAT, N), jnp.float32),
            jax.ShapeDtypeStruct((LAT, N), jnp.float32),
        ],
    )(t2t, s2, u2, disr, bmu, bls)


def kernel(x, edge_index, masked_y, W1, b1, W_mu, b_mu, W_logstd, b_logstd):
    ei = edge_index.astype(jnp.int32)
    histp = _hist_kernel(ei)
    a, cs, csr = _make_a(masked_y)
    dis, disr, s1p = _make_s1(cs, csr, histp, x, W1)
    u1 = _scatter(s1p, ei)
    t1 = _make_t(a, s1p)
    s2p = _make_s2(t1, s1p, u1, dis, b1, W_mu, W_logstd)
    u2 = _scatter(s2p, ei)
    t2t = _make_tT(a, s2p)
    z_mu_t, z_logstd_t = _make_out(t2t, s2p, u2, disr,
                                   b_mu.reshape(LAT, 1), b_logstd.reshape(LAT, 1))
    return (z_mu_t.T, z_logstd_t.T)
